# baseline (speedup 1.0000x reference)
import functools

import jax
import jax.numpy as jnp
from jax import lax
from jax.experimental import pallas as pl
from jax.experimental.pallas import tpu as pltpu


@functools.lru_cache(maxsize=None)
def _make_gather(vocab: int, embed_dim: int, batch: int):
    def body(idx_ref, table_ref, out_ref, sem):
        def loop(i, carry):
            r = idx_ref[i]
            pltpu.make_async_copy(
                table_ref.at[pl.ds(r, 1)], out_ref.at[pl.ds(i, 1)], sem
            ).start()
            return carry

        lax.fori_loop(0, batch, loop, 0, unroll=8)
        pltpu.make_async_copy(
            table_ref.at[pl.ds(0, batch)], out_ref, sem
        ).wait()

    grid_spec = pltpu.PrefetchScalarGridSpec(
        num_scalar_prefetch=1,
        grid=(1,),
        in_specs=[pl.BlockSpec(memory_space=pl.ANY)],
        out_specs=pl.BlockSpec(
            (batch, embed_dim), lambda i, idx_ref: (0, 0)
        ),
        scratch_shapes=[pltpu.SemaphoreType.DMA],
    )
    return pl.pallas_call(
        body,
        grid_spec=grid_spec,
        out_shape=jax.ShapeDtypeStruct((batch, embed_dim), jnp.float32),
    )


def kernel(indices, kernel):
    table = kernel
    vocab, embed_dim = table.shape
    (batch,) = indices.shape
    gather_fn = _make_gather(vocab, embed_dim, batch)
    idx = jnp.asarray(indices, jnp.int32)
    return gather_fn(idx, table)
